# baseline (speedup 1.0000x reference)
import functools

import numpy as np
import jax
import jax.numpy as jnp
from jax import lax
from jax.experimental import pallas as pl
from jax.experimental.pallas import tpu as pltpu
from jax.experimental.pallas import tpu_sc as plsc

N_PTS = 1000000
TBL = 32768
HMASK = 32767
P1 = int(np.uint32(2654435761).view(np.int32))
P2 = 805459861
_p1d = (2 * 2654435761) % (2 ** 32)
P1D = _p1d - 2 ** 32 if _p1d >= 2 ** 31 else _p1d
P2D = 2 * P2
RES = (12.0, 18.0, 27.0, 40.0)
WL = (13, 19, 28)

N0 = WL[0] ** 3
N1 = WL[1] ** 3
N2 = WL[2] ** 3
OFF_D = (0, N0, N0 + N1)
PD = N0 + N1 + N2
OFF_H3 = 2 * PD
OFF_H3P1 = OFF_H3 + TBL
TAB_W = OFF_H3 + 2 * TBL
H3CW = 1024
NHC = 2 * TBL // H3CW

C = 128
XW = C * 3
OW = C * 8
NW = 32
NCH = (N_PTS + C - 1) // C
NCHW = -(-NCH // NW) + (-(-NCH // NW)) % 2
LAST_T = NCH - 1
LAST_X = N_PTS - C
NP_PAD = NCH * C
OUT_W = NP_PAD * 8


@functools.partial(
    pl.kernel,
    mesh=plsc.VectorSubcoreMesh(core_axis_name="c", subcore_axis_name="s"),
    out_type=jax.ShapeDtypeStruct((OUT_W,), jnp.float32),
    compiler_params=pltpu.CompilerParams(needs_layout_passes=False),
    scratch_types=[
        pltpu.VMEM((TAB_W,), jnp.float32),
        pltpu.VMEM((XW,), jnp.float32),
        pltpu.VMEM((XW,), jnp.float32),
        pltpu.VMEM((OW,), jnp.float32),
        pltpu.VMEM((OW,), jnp.float32),
        pltpu.SemaphoreType.DMA,
        pltpu.SemaphoreType.DMA,
        pltpu.SemaphoreType.DMA,
        pltpu.SemaphoreType.DMA,
    ],
)
def _encode_sc(x_hbm, tab_hbm, out_hbm,
               tab, xb0, xb1, ob0, ob1, sx0, sx1, so0, so1):
    wid = lax.axis_index("s") * 2 + lax.axis_index("c")
    iota = lax.iota(jnp.int32, 16)
    iota2 = iota * 2
    xbs, obs = (xb0, xb1), (ob0, ob1)
    sxs, sos = (sx0, sx1), (so0, so1)

    tab_p1 = tab.at[pl.ds(PD, PD)]
    tab_h30 = tab.at[pl.ds(OFF_H3, TBL)]
    tab_h31 = tab.at[pl.ds(OFF_H3P1, TBL)]

    for l in range(3):
        pltpu.sync_copy(tab_hbm.at[pl.ds(l * 2 * TBL, 2 * TBL)],
                        tab.at[pl.ds(OFF_H3, 2 * TBL)])
        w = WL[l]
        n3 = w * w * w

        @plsc.parallel_loop(0, (n3 + 15) // 16, unroll=4)
        def _(g, w=w, n3=n3, l=l):
            m = (2 ** 20 + w - 1) // w
            lin = g * 16 + iota
            q1 = (lin * m) >> 20
            c2 = lin - q1 * w
            c0 = (q1 * m) >> 20
            c1 = q1 - c0 * w
            h = (c0 ^ (c1 * P1) ^ (c2 * P2)) & HMASK
            src = h * 2 + OFF_H3
            tgt = lin + OFF_D[l]
            msk = lin < n3
            v0 = plsc.load_gather(tab, [src])
            v1 = plsc.load_gather(tab, [src + 1])
            plsc.store_scatter(tab, [tgt], v0, mask=msk)
            plsc.store_scatter(tab_p1, [tgt], v1, mask=msk)
    for b in range(2):
        pltpu.async_copy(tab_hbm.at[pl.ds(3 * 2 * TBL + b * H3CW, H3CW)],
                         obs[b], sos[b])

    def h3pair(pi, _):
        for b in range(2):
            k = pi * 2 + b
            buf, so = obs[b], sos[b]
            pltpu.make_async_copy(tab_hbm.at[pl.ds(0, H3CW)], buf, so).wait()

            @plsc.parallel_loop(0, H3CW // 32, unroll=2)
            def _(g, buf=buf, k=k):
                e2 = iota2 + g * 32
                v0 = plsc.load_gather(buf, [e2])
                v1 = plsc.load_gather(buf, [e2 + 1])
                tab[pl.ds(OFF_H3 + k * 512 + g * 16, 16)] = v0
                tab[pl.ds(OFF_H3P1 + k * 512 + g * 16, 16)] = v1

            nk = jnp.minimum(k + 2, NHC - 1)
            pltpu.async_copy(tab_hbm.at[pl.ds(3 * 2 * TBL + nk * H3CW, H3CW)],
                             buf, so)
        return 0

    lax.fori_loop(0, NHC // 2, h3pair, 0)
    for b in range(2):
        pltpu.make_async_copy(tab_hbm.at[pl.ds(0, H3CW)], obs[b], sos[b]).wait()

    def do_group(g, xoff, xb, ob):
        gx = xoff + g * 16
        x0 = xb[pl.ds(jnp.minimum(gx, C - 16), 16)]
        x1 = xb[pl.ds(jnp.minimum(C + gx, 2 * C - 16), 16)]
        x2 = xb[pl.ds(jnp.minimum(2 * C + gx, 3 * C - 16), 16)]
        for l in range(4):
            p0 = x0 * RES[l]
            p1 = x1 * RES[l]
            p2 = x2 * RES[l]
            c0 = p0.astype(jnp.int32)
            c1 = p1.astype(jnp.int32)
            c2 = p2.astype(jnp.int32)
            w0 = p0 - c0.astype(jnp.float32)
            w1 = p1 - c1.astype(jnp.float32)
            w2 = p2 - c2.astype(jnp.float32)
            u0 = 1.0 - w0
            u1 = 1.0 - w1
            u2 = 1.0 - w2
            q = (u0 * u1, u0 * w1, w0 * u1, w0 * w1)
            acc0 = acc1 = None
            if l < 3:
                w = WL[l]
                wb = (c0 * w + c1) * w + c2
                for corner in range(8):
                    b0 = corner & 1
                    b1 = (corner >> 1) & 1
                    b2 = (corner >> 2) & 1
                    coff = OFF_D[l] + b0 * w * w + b1 * w + b2
                    idx = wb + coff if coff else wb
                    g0 = plsc.load_gather(tab, [idx])
                    g1 = plsc.load_gather(tab_p1, [idx])
                    wt = q[b0 * 2 + b1] * (w2 if b2 else u2)
                    acc0 = wt * g0 if acc0 is None else acc0 + wt * g0
                    acc1 = wt * g1 if acc1 is None else acc1 + wt * g1
            else:
                hb = c1 * P1
                hc = c2 * P2
                a1 = c0 + 1
                hb1 = hb + P1
                hc1 = hc + P2
                for corner in range(8):
                    b0 = corner & 1
                    b1 = (corner >> 1) & 1
                    b2 = (corner >> 2) & 1
                    h = ((a1 if b0 else c0)
                         ^ (hb1 if b1 else hb)
                         ^ (hc1 if b2 else hc)) & HMASK
                    g0 = plsc.load_gather(tab_h30, [h])
                    g1 = plsc.load_gather(tab_h31, [h])
                    wt = q[b0 * 2 + b1] * (w2 if b2 else u2)
                    acc0 = wt * g0 if acc0 is None else acc0 + wt * g0
                    acc1 = wt * g1 if acc1 is None else acc1 + wt * g1
            ob[pl.ds(g * 16 + 256 * l, 16)] = acc0
            ob[pl.ds(g * 16 + 256 * l + 128, 16)] = acc1

    def xbase_of(ci):
        ct = jnp.minimum(ci * NW + wid, LAST_T)
        return ct, jnp.minimum(ct * C, LAST_X)

    def start_x(pts, xb, sx):
        for d in range(3):
            pltpu.async_copy(x_hbm.at[pl.ds(d * N_PTS + pts, C)],
                             xb.at[pl.ds(d * C, C)], sx)

    for b in range(2):
        _, xb_pts = xbase_of(b)
        start_x(xb_pts, xbs[b], sxs[b])

    def chunk_pair(pi, _):
        for b in range(2):
            ci = pi * 2 + b
            xb, ob, sx, so = xbs[b], obs[b], sxs[b], sos[b]
            ct, xb_pts = xbase_of(ci)
            xoff = ct * C - xb_pts
            pltpu.make_async_copy(x_hbm.at[pl.ds(0, XW)], xb, sx).wait()

            @pl.when(pi > 0)
            def _():
                pltpu.make_async_copy(ob, out_hbm.at[pl.ds(0, OW)], so).wait()

            @plsc.parallel_loop(0, C // 16, unroll=2)
            def _(g, xoff=xoff, xb=xb, ob=ob):
                do_group(g, xoff, xb, ob)

            pltpu.async_copy(ob, out_hbm.at[pl.ds(ct * OW, OW)], so)
            _, nxt = xbase_of(ci + 2)
            start_x(nxt, xb, sx)
        return 0

    lax.fori_loop(0, NCHW // 2, chunk_pair, 0)
    for b in range(2):
        pltpu.make_async_copy(x_hbm.at[pl.ds(0, XW)], xbs[b], sxs[b]).wait()
        pltpu.make_async_copy(obs[b], out_hbm.at[pl.ds(0, OW)], sos[b]).wait()


def kernel(x, table):
    out = _encode_sc(x.T.reshape(-1), table.reshape(-1))
    out = out.reshape(NCH, 8, C).transpose(0, 2, 1).reshape(NP_PAD, 8)
    return out[:N_PTS]


# speedup vs baseline: 1.0360x; 1.0360x over previous
import functools

import numpy as np
import jax
import jax.numpy as jnp
from jax import lax
from jax.experimental import pallas as pl
from jax.experimental.pallas import tpu as pltpu
from jax.experimental.pallas import tpu_sc as plsc

N_PTS = 1000000
TBL = 32768
HMASK = 32767
P1 = int(np.uint32(2654435761).view(np.int32))
P2 = 805459861
_p1d = (2 * 2654435761) % (2 ** 32)
P1D = _p1d - 2 ** 32 if _p1d >= 2 ** 31 else _p1d
P2D = 2 * P2
RES = (12.0, 18.0, 27.0, 40.0)
WL = (13, 19, 28)

N0 = WL[0] ** 3
N1 = WL[1] ** 3
N2 = WL[2] ** 3
OFF_D = (0, N0, N0 + N1)
PD = N0 + N1 + N2
OFF_H3 = 2 * PD
TAB_W = OFF_H3 + 2 * TBL

C = 128
XW = C * 3
OW = C * 8
NW = 32
NCH = (N_PTS + C - 1) // C
NCHW = -(-NCH // NW) + (-(-NCH // NW)) % 2
LAST_T = NCH - 1
LAST_X = N_PTS - C
NP_PAD = NCH * C
OUT_W = NP_PAD * 8


@functools.partial(
    pl.kernel,
    mesh=plsc.VectorSubcoreMesh(core_axis_name="c", subcore_axis_name="s"),
    out_type=jax.ShapeDtypeStruct((OUT_W,), jnp.float32),
    compiler_params=pltpu.CompilerParams(needs_layout_passes=False),
    scratch_types=[
        pltpu.VMEM((TAB_W,), jnp.float32),
        pltpu.VMEM((XW,), jnp.float32),
        pltpu.VMEM((XW,), jnp.float32),
        pltpu.VMEM((OW,), jnp.float32),
        pltpu.VMEM((OW,), jnp.float32),
        pltpu.SemaphoreType.DMA,
        pltpu.SemaphoreType.DMA,
        pltpu.SemaphoreType.DMA,
        pltpu.SemaphoreType.DMA,
    ],
)
def _encode_sc(x_hbm, tab_hbm, out_hbm,
               tab, xb0, xb1, ob0, ob1, sx0, sx1, so0, so1):
    wid = lax.axis_index("s") * 2 + lax.axis_index("c")
    iota = lax.iota(jnp.int32, 16)
    iota3 = iota * 3
    iota8 = iota * 8
    xbs, obs = (xb0, xb1), (ob0, ob1)
    sxs, sos = (sx0, sx1), (so0, so1)

    tab_p1 = tab.at[pl.ds(PD, PD)]
    tab_h3 = tab.at[pl.ds(OFF_H3, 2 * TBL)]

    for l in range(3):
        pltpu.sync_copy(tab_hbm.at[pl.ds(l * 2 * TBL, 2 * TBL)],
                        tab.at[pl.ds(OFF_H3, 2 * TBL)])
        w = WL[l]
        n3 = w * w * w

        @plsc.parallel_loop(0, (n3 + 15) // 16, unroll=4)
        def _(g, w=w, n3=n3, l=l):
            m = (2 ** 20 + w - 1) // w
            lin = g * 16 + iota
            q1 = (lin * m) >> 20
            c2 = lin - q1 * w
            c0 = (q1 * m) >> 20
            c1 = q1 - c0 * w
            h = (c0 ^ (c1 * P1) ^ (c2 * P2)) & HMASK
            src = h * 2 + OFF_H3
            tgt = lin + OFF_D[l]
            msk = lin < n3
            v0 = plsc.load_gather(tab, [src])
            v1 = plsc.load_gather(tab, [src + 1])
            plsc.store_scatter(tab, [tgt], v0, mask=msk)
            plsc.store_scatter(tab_p1, [tgt], v1, mask=msk)
    pltpu.sync_copy(tab_hbm.at[pl.ds(3 * 2 * TBL, 2 * TBL)],
                    tab.at[pl.ds(OFF_H3, 2 * TBL)])

    def do_group(g, xoff, xb, ob):
        gx = xoff + g * 16
        x0 = xb[pl.ds(jnp.minimum(gx, C - 16), 16)]
        x1 = xb[pl.ds(jnp.minimum(C + gx, 2 * C - 16), 16)]
        x2 = xb[pl.ds(jnp.minimum(2 * C + gx, 3 * C - 16), 16)]
        for l in range(4):
            p0 = x0 * RES[l]
            p1 = x1 * RES[l]
            p2 = x2 * RES[l]
            c0 = p0.astype(jnp.int32)
            c1 = p1.astype(jnp.int32)
            c2 = p2.astype(jnp.int32)
            w0 = p0 - c0.astype(jnp.float32)
            w1 = p1 - c1.astype(jnp.float32)
            w2 = p2 - c2.astype(jnp.float32)
            u0 = 1.0 - w0
            u1 = 1.0 - w1
            u2 = 1.0 - w2
            q = (u0 * u1, u0 * w1, w0 * u1, w0 * w1)
            acc0 = acc1 = None
            if l < 3:
                w = WL[l]
                wb = (c0 * w + c1) * w + c2
                for corner in range(8):
                    b0 = corner & 1
                    b1 = (corner >> 1) & 1
                    b2 = (corner >> 2) & 1
                    coff = OFF_D[l] + b0 * w * w + b1 * w + b2
                    idx = wb + coff if coff else wb
                    g0 = plsc.load_gather(tab, [idx])
                    g1 = plsc.load_gather(tab_p1, [idx])
                    wt = q[b0 * 2 + b1] * (w2 if b2 else u2)
                    acc0 = wt * g0 if acc0 is None else acc0 + wt * g0
                    acc1 = wt * g1 if acc1 is None else acc1 + wt * g1
            else:
                ha = c0 + c0
                hb = c1 * P1D
                hc = c2 * P2D
                ha1 = ha + 2
                hb1 = hb + P1D
                hc1 = hc + P2D
                for corner in range(8):
                    b0 = corner & 1
                    b1 = (corner >> 1) & 1
                    b2 = (corner >> 2) & 1
                    idx = ((ha1 if b0 else ha)
                           ^ (hb1 if b1 else hb)
                           ^ (hc1 if b2 else hc)) & (2 * HMASK)
                    g0 = plsc.load_gather(tab_h3, [idx])
                    g1 = plsc.load_gather(tab_h3, [idx + 1])
                    wt = q[b0 * 2 + b1] * (w2 if b2 else u2)
                    acc0 = wt * g0 if acc0 is None else acc0 + wt * g0
                    acc1 = wt * g1 if acc1 is None else acc1 + wt * g1
            ob[pl.ds(g * 16 + 256 * l, 16)] = acc0
            ob[pl.ds(g * 16 + 256 * l + 128, 16)] = acc1

    def xbase_of(ci):
        ct = jnp.minimum(ci * NW + wid, LAST_T)
        return ct, jnp.minimum(ct * C, LAST_X)

    def start_x(pts, xb, sx):
        for d in range(3):
            pltpu.async_copy(x_hbm.at[pl.ds(d * N_PTS + pts, C)],
                             xb.at[pl.ds(d * C, C)], sx)

    for b in range(2):
        _, xb_pts = xbase_of(b)
        start_x(xb_pts, xbs[b], sxs[b])

    def chunk_pair(pi, _):
        for b in range(2):
            ci = pi * 2 + b
            xb, ob, sx, so = xbs[b], obs[b], sxs[b], sos[b]
            ct, xb_pts = xbase_of(ci)
            xoff = ct * C - xb_pts
            pltpu.make_async_copy(x_hbm.at[pl.ds(0, XW)], xb, sx).wait()

            @pl.when(pi > 0)
            def _():
                pltpu.make_async_copy(ob, out_hbm.at[pl.ds(0, OW)], so).wait()

            @plsc.parallel_loop(0, C // 16, unroll=2)
            def _(g, xoff=xoff, xb=xb, ob=ob):
                do_group(g, xoff, xb, ob)

            pltpu.async_copy(ob, out_hbm.at[pl.ds(ct * OW, OW)], so)
            _, nxt = xbase_of(ci + 2)
            start_x(nxt, xb, sx)
        return 0

    lax.fori_loop(0, NCHW // 2, chunk_pair, 0)
    for b in range(2):
        pltpu.make_async_copy(x_hbm.at[pl.ds(0, XW)], xbs[b], sxs[b]).wait()
        pltpu.make_async_copy(obs[b], out_hbm.at[pl.ds(0, OW)], sos[b]).wait()


def kernel(x, table):
    out = _encode_sc(x.T.reshape(-1), table.reshape(-1))
    out = out.reshape(NCH, 8, C).transpose(0, 2, 1).reshape(NP_PAD, 8)
    return out[:N_PTS]
